# baseline (speedup 1.0000x reference)
import jax
import jax.numpy as jnp
from jax.experimental import pallas as pl

_H = 8
_KVH = 2
_GS = _H // _KVH
_QKB = 8
_VB = 16
_TAU = 1.0
_BQ = 256
_HB = _BQ // 2

_LOG2E = 1.4426950408889634


def _nt_dot(a, b):
    return jax.lax.dot_general(a, b, (((1,), (1,)), ((), ())),
                               preferred_element_type=jnp.float32)


def _rosa_kernel(hs_ref, wq_ref, wk_ref, wv_ref, wo_ref, ve0_ref, ve1_ref,
                 out_ref):
    S = hs_ref.shape[0]
    nq = S // _BQ
    bf = jnp.bfloat16

    hs = hs_ref[...].astype(bf)
    qd = jnp.dot(hs, wq_ref[...].astype(bf),
                 preferred_element_type=jnp.float32)
    kd = jnp.dot(hs, wk_ref[...].astype(bf),
                 preferred_element_type=jnp.float32)
    vd = jnp.dot(hs, wv_ref[...].astype(bf),
                 preferred_element_type=jnp.float32)
    qp = jax.nn.sigmoid(qd / _TAU)
    kb = jax.nn.sigmoid(kd / _TAU)
    vb = jax.nn.sigmoid(vd / _TAU)

    ones = jnp.ones((S, 1), jnp.float32)
    z6 = jnp.zeros((S, 16 - _QKB - 2), jnp.float32)
    z15 = jnp.zeros((S, 32 - _VB - 1), jnp.float32)
    qas = [jnp.concatenate(
        [qp[:, h * _QKB:(h + 1) * _QKB], ones, ones, z6],
        axis=1).astype(bf) for h in range(_H)]
    kas, vas = [], []
    for g in range(_KVH):
        kbg = kb[:, g * _QKB:(g + 1) * _QKB]
        bias = -_LOG2E * jnp.sum(kbg, axis=1, keepdims=True)
        bias_hi = bias.astype(bf).astype(jnp.float32)
        bias_lo = bias - bias_hi
        kas.append(jnp.concatenate(
            [(2.0 * _LOG2E) * kbg, bias_hi, bias_lo, z6], axis=1).astype(bf))
        vas.append(jnp.concatenate(
            [vb[:, g * _VB:(g + 1) * _VB], ones, z15], axis=1))

    dmask_l = (jax.lax.broadcasted_iota(jnp.int32, (_BQ, _HB), 1)
               <= jax.lax.broadcasted_iota(jnp.int32, (_BQ, _HB), 0))
    dmask_h = (jax.lax.broadcasted_iota(jnp.int32, (_HB, _HB), 1)
               <= jax.lax.broadcasted_iota(jnp.int32, (_HB, _HB), 0))
    zpad_top = jnp.zeros((_HB, 32), jnp.float32)

    vmix_blocks = []
    for qi in range(nq):
        r0, r1 = qi * _BQ, (qi + 1) * _BQ
        obits = []
        for h in range(_H):
            g = h // _GS
            qh = qas[h][r0:r1]
            acc = None
            for j in range(qi):
                c0, c1 = j * _BQ, (j + 1) * _BQ
                p = jnp.exp2(_nt_dot(qh, kas[g][c0:c1]))
                o = jnp.dot(p, vas[g][c0:c1],
                            preferred_element_type=jnp.float32)
                acc = o if acc is None else acc + o
            kg, vg = kas[g][r0:r1], vas[g][r0:r1]
            p_l = jnp.where(dmask_l, jnp.exp2(_nt_dot(qh, kg[:_HB])), 0.0)
            o = jnp.dot(p_l, vg[:_HB], preferred_element_type=jnp.float32)
            p_br = jnp.where(dmask_h,
                             jnp.exp2(_nt_dot(qh[_HB:], kg[_HB:])), 0.0)
            o_br = jnp.dot(p_br, vg[_HB:], preferred_element_type=jnp.float32)
            o = o + jnp.concatenate([zpad_top, o_br], axis=0)
            acc = o if acc is None else acc + o
            obits.append(acc[:, :_VB] / acc[:, _VB:_VB + 1])
        ob = jnp.concatenate(obits, axis=1)
        vmix_blocks.append(
            (ve0_ref[...] * (1.0 - ob) + ve1_ref[...] * ob).astype(bf))
    vmix = jnp.concatenate(vmix_blocks, axis=0)
    out_ref[...] = jnp.dot(vmix, wo_ref[...].astype(bf),
                           preferred_element_type=jnp.float32)


def _rosa_single(hs, Wq, Wk, Wv, Wo, ve0, ve1, interpret=False):
    S, HID = hs.shape
    return pl.pallas_call(
        _rosa_kernel,
        out_shape=jax.ShapeDtypeStruct((S, HID), jnp.float32),
        interpret=interpret,
    )(hs, Wq, Wk, Wv, Wo, ve0.reshape(1, -1), ve1.reshape(1, -1))


def kernel(hidden_states, Wq, Wk, Wv, Wo, v_emb0, v_emb1):
    B = hidden_states.shape[0]
    outs = [_rosa_single(hidden_states[b], Wq, Wk, Wv, Wo, v_emb0, v_emb1)
            for b in range(B)]
    if B == 1:
        return jnp.expand_dims(outs[0], 0)
    return jnp.stack(outs, axis=0)


# speedup vs baseline: 1.1350x; 1.1350x over previous
import jax
import jax.numpy as jnp
from jax.experimental import pallas as pl

_H = 8
_KVH = 2
_GS = _H // _KVH
_QKB = 8
_VB = 16
_TAU = 1.0
_BQ = 256

_LOG2E = 1.4426950408889634


def _nt_dot(a, b):
    return jax.lax.dot_general(a, b, (((1,), (1,)), ((), ())),
                               preferred_element_type=jnp.float32)


def _rosa_kernel(hs_ref, wq_ref, wk_ref, wv_ref, wo_ref, ve0_ref, ve1_ref,
                 out_ref):
    S = hs_ref.shape[0]
    nq = S // _BQ
    bf = jnp.bfloat16

    hs = hs_ref[...].astype(bf)
    qd = jnp.dot(hs, wq_ref[...].astype(bf),
                 preferred_element_type=jnp.float32)
    kd = jnp.dot(hs, wk_ref[...].astype(bf),
                 preferred_element_type=jnp.float32)
    vd = jnp.dot(hs, wv_ref[...].astype(bf),
                 preferred_element_type=jnp.float32)
    qp = jax.nn.sigmoid(qd / _TAU)
    kb = jax.nn.sigmoid(kd / _TAU)
    vb = jax.nn.sigmoid(vd / _TAU)

    ones = jnp.ones((S, 1), jnp.float32)
    z6 = jnp.zeros((S, 16 - _QKB - 2), jnp.float32)
    z15 = jnp.zeros((S, 32 - _VB - 1), jnp.float32)
    qas = [jnp.concatenate(
        [qp[:, h * _QKB:(h + 1) * _QKB], ones, ones, z6],
        axis=1).astype(bf) for h in range(_H)]
    kas, vas = [], []
    for g in range(_KVH):
        kbg = kb[:, g * _QKB:(g + 1) * _QKB]
        bias = -_LOG2E * jnp.sum(kbg, axis=1, keepdims=True)
        bias_hi = bias.astype(bf).astype(jnp.float32)
        bias_lo = bias - bias_hi
        kas.append(jnp.concatenate(
            [(2.0 * _LOG2E) * kbg, bias_hi, bias_lo, z6], axis=1).astype(bf))
        vas.append(jnp.concatenate(
            [vb[:, g * _VB:(g + 1) * _VB], ones, z15], axis=1))

    dmask = (jax.lax.broadcasted_iota(jnp.int32, (_BQ, _BQ), 1)
             <= jax.lax.broadcasted_iota(jnp.int32, (_BQ, _BQ), 0))

    vmix_blocks = []
    for qi in range(nq):
        r0, r1 = qi * _BQ, (qi + 1) * _BQ
        obits = []
        for h in range(_H):
            g = h // _GS
            qh = qas[h][r0:r1]
            acc = None
            for j in range(qi + 1):
                c0, c1 = j * _BQ, (j + 1) * _BQ
                p = jnp.exp2(_nt_dot(qh, kas[g][c0:c1]))
                if j == qi:
                    p = jnp.where(dmask, p, 0.0)
                o = jnp.dot(p, vas[g][c0:c1],
                            preferred_element_type=jnp.float32)
                acc = o if acc is None else acc + o
            obits.append(acc[:, :_VB] / acc[:, _VB:_VB + 1])
        ob = jnp.concatenate(obits, axis=1)
        vmix_blocks.append(
            (ve0_ref[...] * (1.0 - ob) + ve1_ref[...] * ob).astype(bf))
    vmix = jnp.concatenate(vmix_blocks, axis=0)
    out_ref[...] = jnp.dot(vmix, wo_ref[...].astype(bf),
                           preferred_element_type=jnp.float32)


def _rosa_single(hs, Wq, Wk, Wv, Wo, ve0, ve1, interpret=False):
    S, HID = hs.shape
    return pl.pallas_call(
        _rosa_kernel,
        out_shape=jax.ShapeDtypeStruct((S, HID), jnp.float32),
        interpret=interpret,
    )(hs, Wq, Wk, Wv, Wo, ve0.reshape(1, -1), ve1.reshape(1, -1))


def kernel(hidden_states, Wq, Wk, Wv, Wo, v_emb0, v_emb1):
    B = hidden_states.shape[0]
    outs = [_rosa_single(hidden_states[b], Wq, Wk, Wv, Wo, v_emb0, v_emb1)
            for b in range(B)]
    if B == 1:
        return jnp.expand_dims(outs[0], 0)
    return jnp.stack(outs, axis=0)
